# baseline (speedup 1.0000x reference)
import functools

import jax
import jax.numpy as jnp
from jax import lax
from jax.experimental import pallas as pl
from jax.experimental.pallas import tpu as pltpu
from jax.experimental.pallas import tpu_sc as plsc

B = 16384
D = 64
HIDDEN = 128
NC = 2
NS = 16
NW = NC * NS
BPW = B // NW
CH = 128
NCH = BPW // CH


def _gather_sc(idx2, item_table, user_table):
    mesh = plsc.VectorSubcoreMesh(
        core_axis_name="c", subcore_axis_name="s",
        num_cores=NC, num_subcores=NS)

    @functools.partial(
        pl.kernel,
        out_type=(
            jax.ShapeDtypeStruct((B, D), jnp.float32),
            jax.ShapeDtypeStruct((B, D), jnp.float32),
        ),
        mesh=mesh,
        scratch_types=[
            pltpu.VMEM((2 * NCH, CH), jnp.int32),
            pltpu.VMEM((BPW, D), jnp.float32),
            pltpu.VMEM((BPW, D), jnp.float32),
            pltpu.SemaphoreType.DMA,
        ],
        compiler_params=pltpu.CompilerParams(use_tc_tiling_on_sc=False),
    )
    def k(idx_hbm, item_hbm, user_hbm, y1_hbm, y2_hbm, idx_v, rows_i, rows_u,
          sem):
        wid = lax.axis_index("s") * NC + lax.axis_index("c")
        base = wid * BPW
        pltpu.sync_copy(idx_hbm.at[wid], idx_v)
        copies = []
        for j in range(NCH):
            copies.append(pltpu.async_copy(
                item_hbm.at[idx_v.at[j]],
                rows_i.at[pl.ds(j * CH, CH)], sem))
        for j in range(NCH):
            copies.append(pltpu.async_copy(
                user_hbm.at[idx_v.at[NCH + j]],
                rows_u.at[pl.ds(j * CH, CH)], sem))
        for c in copies:
            c.wait()
        pltpu.sync_copy(rows_i, y1_hbm.at[pl.ds(base, BPW)])
        pltpu.sync_copy(rows_u, y2_hbm.at[pl.ds(base, BPW)])

    return k(idx2, item_table, user_table)


def _mlp_tc(y1, y2, W1, b1r, w2r, b2s):
    BB = 2048

    def body(y1_ref, y2_ref, w1_ref, b1_ref, w2_ref, b2_ref, o_ref):
        h = jnp.dot(y1_ref[...], w1_ref[0:D, :],
                    preferred_element_type=jnp.float32)
        h = h + jnp.dot(y2_ref[...], w1_ref[D:2 * D, :],
                        preferred_element_type=jnp.float32)
        h = h + b1_ref[...]
        h = jnp.maximum(h, 0.0)
        o = jnp.sum(h * w2_ref[...], axis=1, keepdims=True) + b2_ref[0]
        o_ref[...] = jax.nn.sigmoid(o)

    return pl.pallas_call(
        body,
        grid=(B // BB,),
        in_specs=[
            pl.BlockSpec((BB, D), lambda i: (i, 0)),
            pl.BlockSpec((BB, D), lambda i: (i, 0)),
            pl.BlockSpec((2 * D, HIDDEN), lambda i: (0, 0)),
            pl.BlockSpec((1, HIDDEN), lambda i: (0, 0)),
            pl.BlockSpec((1, HIDDEN), lambda i: (0, 0)),
            pl.BlockSpec(memory_space=pltpu.SMEM),
        ],
        out_specs=pl.BlockSpec((BB, 1), lambda i: (i, 0)),
        out_shape=jax.ShapeDtypeStruct((B, 1), jnp.float32),
    )(y1, y2, W1, b1r, w2r, b2s)


def kernel(user_id, input_ids, item_table, user_table, W1, b1, W2, b2):
    ii = input_ids.astype(jnp.int32).reshape(NW, NCH, CH)
    ui = user_id.astype(jnp.int32).reshape(NW, NCH, CH)
    idx2 = jnp.concatenate([ii, ui], axis=1)
    y1, y2 = _gather_sc(idx2, item_table, user_table)
    return _mlp_tc(y1, y2, W1, b1.reshape(1, HIDDEN),
                   W2.reshape(1, HIDDEN), b2)


# speedup vs baseline: 2.6729x; 2.6729x over previous
import functools

import jax
import jax.numpy as jnp
from jax import lax
from jax.experimental import pallas as pl
from jax.experimental.pallas import tpu as pltpu
from jax.experimental.pallas import tpu_sc as plsc

B = 16384
D = 64
HIDDEN = 128
NC = 2
NS = 16
NW = NC * NS
BPW = B // NW
G = 16
NG = BPW // G
RING = 4


def _gather_sc(itemT, userT, input_ids, user_id):
    mesh = plsc.VectorSubcoreMesh(
        core_axis_name="c", subcore_axis_name="s",
        num_cores=NC, num_subcores=NS)

    @functools.partial(
        pl.kernel,
        out_type=jax.ShapeDtypeStruct((B, 2 * D), jnp.float32),
        mesh=mesh,
        scratch_types=[
            pltpu.VMEM((BPW,), jnp.int32),
            pltpu.VMEM((BPW,), jnp.int32),
            pltpu.VMEM((RING, D, 128), jnp.float32),
            pltpu.VMEM((RING, D, 128), jnp.float32),
            pltpu.VMEM((G, 2 * D), jnp.float32),
            pltpu.VMEM((G,), jnp.int32),
            [pltpu.SemaphoreType.DMA] * RING,
            [pltpu.SemaphoreType.DMA] * RING,
            pltpu.SemaphoreType.DMA,
            pltpu.SemaphoreType.DMA,
        ],
        compiler_params=pltpu.CompilerParams(needs_layout_passes=False),
    )
    def k(itemT_hbm, userT_hbm, iid_hbm, uid_hbm, out_hbm,
          iid_v, uid_v, ring_i, ring_u, stage, bidx, sems_i, sems_u,
          sem_ids, sem_sc):
        wid = lax.axis_index("s") * NC + lax.axis_index("c")
        base = wid * BPW
        pltpu.async_copy(iid_hbm.at[pl.ds(base, BPW)], iid_v, sem_ids).wait()
        pltpu.async_copy(uid_hbm.at[pl.ds(base, BPW)], uid_v, sem_ids).wait()

        lanes = lax.broadcasted_iota(jnp.int32, (16,), 0)

        def fire(tbl, ring, sems, idx_scalar, slot):
            cb = pl.multiple_of((idx_scalar // 128) * 128, 128)
            return pltpu.async_copy(
                tbl.at[:, pl.ds(cb, 128)], ring.at[slot], sems[slot])

        def extract(ring, vec_j, slot, row, col0):
            l = vec_j % 128
            coli = jnp.full((16,), 0, jnp.int32) + l
            for q in range(D // 16):
                rowi = q * 16 + lanes
                vals = plsc.load_gather(ring.at[slot], [rowi, coli])
                stage[row, pl.ds(col0 + q * 16, 16)] = vals

        vi0 = iid_v[pl.ds(0, G)]
        vu0 = uid_v[pl.ds(0, G)]
        for j in range(RING):
            fire(itemT_hbm, ring_i, sems_i, vi0[j], j)
            fire(userT_hbm, ring_u, sems_u, vu0[j], j)

        def group(g, carry):
            vi = iid_v[pl.ds(g * G, G)]
            vu = uid_v[pl.ds(g * G, G)]
            gn = jnp.minimum(g + 1, NG - 1)
            vin = iid_v[pl.ds(gn * G, G)]
            vun = uid_v[pl.ds(gn * G, G)]
            for j in range(G):
                slot = j % RING
                pltpu.make_async_copy(
                    itemT_hbm.at[:, pl.ds(0, 128)], ring_i.at[slot],
                    sems_i[slot]).wait()
                pltpu.make_async_copy(
                    userT_hbm.at[:, pl.ds(0, 128)], ring_u.at[slot],
                    sems_u[slot]).wait()
                extract(ring_i, vi[j], slot, j, 0)
                extract(ring_u, vu[j], slot, j, D)
                if j < G - RING:
                    fire(itemT_hbm, ring_i, sems_i, vi[j + RING], slot)
                    fire(userT_hbm, ring_u, sems_u, vu[j + RING], slot)
                else:
                    fire(itemT_hbm, ring_i, sems_i, vin[j + RING - G], slot)
                    fire(userT_hbm, ring_u, sems_u, vun[j + RING - G], slot)
            bidx[...] = lanes + (base + g * G)
            pltpu.async_copy(stage, out_hbm.at[bidx], sem_sc).wait()
            return carry

        lax.fori_loop(0, NG, group, 0)
        for j in range(RING):
            pltpu.make_async_copy(
                itemT_hbm.at[:, pl.ds(0, 128)], ring_i.at[j], sems_i[j]).wait()
            pltpu.make_async_copy(
                userT_hbm.at[:, pl.ds(0, 128)], ring_u.at[j], sems_u[j]).wait()

    return k(itemT, userT, input_ids, user_id)


def _mlp_tc(g, W1, b1r, w2r, b2s):
    BB = 2048

    def body(g_ref, w1_ref, b1_ref, w2_ref, b2_ref, o_ref):
        h = jnp.dot(g_ref[...], w1_ref[...], preferred_element_type=jnp.float32)
        h = h + b1_ref[...]
        h = jnp.maximum(h, 0.0)
        o = jnp.sum(h * w2_ref[...], axis=1, keepdims=True) + b2_ref[0]
        o_ref[...] = jax.nn.sigmoid(o)

    return pl.pallas_call(
        body,
        grid=(B // BB,),
        in_specs=[
            pl.BlockSpec((BB, 2 * D), lambda i: (i, 0)),
            pl.BlockSpec((2 * D, HIDDEN), lambda i: (0, 0)),
            pl.BlockSpec((1, HIDDEN), lambda i: (0, 0)),
            pl.BlockSpec((1, HIDDEN), lambda i: (0, 0)),
            pl.BlockSpec(memory_space=pltpu.SMEM),
        ],
        out_specs=pl.BlockSpec((BB, 1), lambda i: (i, 0)),
        out_shape=jax.ShapeDtypeStruct((B, 1), jnp.float32),
    )(g, W1, b1r, w2r, b2s)


def kernel(user_id, input_ids, item_table, user_table, W1, b1, W2, b2):
    g = _gather_sc(item_table.T, user_table.T,
                   input_ids.astype(jnp.int32), user_id.astype(jnp.int32))
    return _mlp_tc(g, W1, b1.reshape(1, HIDDEN), W2.reshape(1, HIDDEN), b2)


# speedup vs baseline: 2.7903x; 1.0439x over previous
import functools

import jax
import jax.numpy as jnp
from jax import lax
from jax.experimental import pallas as pl
from jax.experimental.pallas import tpu as pltpu
from jax.experimental.pallas import tpu_sc as plsc

B = 16384
D = 64
HIDDEN = 128
NC = 2
NS = 16
NW = NC * NS
BPW = B // NW
G = 16
NG = BPW // G
RING = 6


def _gather_sc(itemT, userT, input_ids, user_id):
    mesh = plsc.VectorSubcoreMesh(
        core_axis_name="c", subcore_axis_name="s",
        num_cores=NC, num_subcores=NS)

    @functools.partial(
        pl.kernel,
        out_type=jax.ShapeDtypeStruct((B, 2 * D), jnp.float32),
        mesh=mesh,
        scratch_types=[
            pltpu.VMEM((BPW,), jnp.int32),
            pltpu.VMEM((BPW,), jnp.int32),
            pltpu.VMEM((RING, D, 128), jnp.float32),
            pltpu.VMEM((RING, D, 128), jnp.float32),
            pltpu.VMEM((G, 2 * D), jnp.float32),
            pltpu.VMEM((G,), jnp.int32),
            [pltpu.SemaphoreType.DMA] * RING,
            [pltpu.SemaphoreType.DMA] * RING,
            pltpu.SemaphoreType.DMA,
            pltpu.SemaphoreType.DMA,
        ],
        compiler_params=pltpu.CompilerParams(needs_layout_passes=False),
    )
    def k(itemT_hbm, userT_hbm, iid_hbm, uid_hbm, out_hbm,
          iid_v, uid_v, ring_i, ring_u, stage, bidx, sems_i, sems_u,
          sem_ids, sem_sc):
        wid = lax.axis_index("s") * NC + lax.axis_index("c")
        base = wid * BPW
        pltpu.async_copy(iid_hbm.at[pl.ds(base, BPW)], iid_v, sem_ids).wait()
        pltpu.async_copy(uid_hbm.at[pl.ds(base, BPW)], uid_v, sem_ids).wait()

        lanes = lax.broadcasted_iota(jnp.int32, (16,), 0)

        def fire(tbl, ring, sems, idx_scalar, slot):
            cb = pl.multiple_of((idx_scalar // 128) * 128, 128)
            return pltpu.async_copy(
                tbl.at[:, pl.ds(cb, 128)], ring.at[slot], sems[slot])

        def extract(ring, vec_j, slot, row, col0):
            l = vec_j % 128
            coli = jnp.full((16,), 0, jnp.int32) + l
            for q in range(D // 16):
                rowi = q * 16 + lanes
                vals = plsc.load_gather(ring.at[slot], [rowi, coli])
                stage[row, pl.ds(col0 + q * 16, 16)] = vals

        vi0 = iid_v[pl.ds(0, G)]
        vu0 = uid_v[pl.ds(0, G)]
        for j in range(RING):
            fire(itemT_hbm, ring_i, sems_i, vi0[j], j)
            fire(userT_hbm, ring_u, sems_u, vu0[j], j)

        def group(g, carry):
            vi = iid_v[pl.ds(g * G, G)]
            vu = uid_v[pl.ds(g * G, G)]
            gn = jnp.minimum(g + 1, NG - 1)
            vin = iid_v[pl.ds(gn * G, G)]
            vun = uid_v[pl.ds(gn * G, G)]
            for j in range(G):
                slot = j % RING
                pltpu.make_async_copy(
                    itemT_hbm.at[:, pl.ds(0, 128)], ring_i.at[slot],
                    sems_i[slot]).wait()
                pltpu.make_async_copy(
                    userT_hbm.at[:, pl.ds(0, 128)], ring_u.at[slot],
                    sems_u[slot]).wait()
                extract(ring_i, vi[j], slot, j, 0)
                extract(ring_u, vu[j], slot, j, D)
                if j < G - RING:
                    fire(itemT_hbm, ring_i, sems_i, vi[j + RING], slot)
                    fire(userT_hbm, ring_u, sems_u, vu[j + RING], slot)
                else:
                    fire(itemT_hbm, ring_i, sems_i, vin[j + RING - G], slot)
                    fire(userT_hbm, ring_u, sems_u, vun[j + RING - G], slot)
            bidx[...] = lanes + (base + g * G)
            pltpu.async_copy(stage, out_hbm.at[bidx], sem_sc).wait()
            return carry

        lax.fori_loop(0, NG, group, 0)
        for j in range(RING):
            pltpu.make_async_copy(
                itemT_hbm.at[:, pl.ds(0, 128)], ring_i.at[j], sems_i[j]).wait()
            pltpu.make_async_copy(
                userT_hbm.at[:, pl.ds(0, 128)], ring_u.at[j], sems_u[j]).wait()

    return k(itemT, userT, input_ids, user_id)


def _mlp_tc(g, W1, b1r, w2r, b2s):
    BB = 2048

    def body(g_ref, w1_ref, b1_ref, w2_ref, b2_ref, o_ref):
        h = jnp.dot(g_ref[...], w1_ref[...], preferred_element_type=jnp.float32)
        h = h + b1_ref[...]
        h = jnp.maximum(h, 0.0)
        o = jnp.sum(h * w2_ref[...], axis=1, keepdims=True) + b2_ref[0]
        o_ref[...] = jax.nn.sigmoid(o)

    return pl.pallas_call(
        body,
        grid=(B // BB,),
        in_specs=[
            pl.BlockSpec((BB, 2 * D), lambda i: (i, 0)),
            pl.BlockSpec((2 * D, HIDDEN), lambda i: (0, 0)),
            pl.BlockSpec((1, HIDDEN), lambda i: (0, 0)),
            pl.BlockSpec((1, HIDDEN), lambda i: (0, 0)),
            pl.BlockSpec(memory_space=pltpu.SMEM),
        ],
        out_specs=pl.BlockSpec((BB, 1), lambda i: (i, 0)),
        out_shape=jax.ShapeDtypeStruct((B, 1), jnp.float32),
    )(g, W1, b1r, w2r, b2s)


def kernel(user_id, input_ids, item_table, user_table, W1, b1, W2, b2):
    g = _gather_sc(item_table.T, user_table.T,
                   input_ids.astype(jnp.int32), user_id.astype(jnp.int32))
    return _mlp_tc(g, W1, b1.reshape(1, HIDDEN), W2.reshape(1, HIDDEN), b2)
